# baseline (speedup 1.0000x reference)
import dataclasses

import jax
import jax.numpy as jnp
from jax import lax
from jax.experimental import pallas as pl
from jax.experimental.pallas import tpu as pltpu
from jax.experimental.pallas import tpu_sc as plsc

_CPARAMS = pltpu.CompilerParams()
if "needs_layout_passes" in pltpu.CompilerParams.__dataclass_fields__:
    _CPARAMS = dataclasses.replace(_CPARAMS, needs_layout_passes=False)

ROWS = 1024
COLS = 8192
NUM_OUT = 65536
L = 16
NC = 2
NS = 16
NW = NC * NS
ROWS_PER_W = ROWS // NW
GROUPS = 256
CHUNKS = 4
NBUF = 4
CHUNK_G = GROUPS // CHUNKS
CHUNK_WORDS = CHUNK_G * GROUPS
GQ = 4


def _sc_body(x_hbm, out_hbm, xrow0, xrow1, ob0, ob1, ob2, ob3,
             isem0, isem1, osem):
    obs = (ob0, ob1, ob2, ob3)
    cid = lax.axis_index("c")
    sid = lax.axis_index("s")
    wid = sid * NC + cid
    row0 = wid * ROWS_PER_W

    idx_splats = [jnp.full((L, 1), i, jnp.int32) for i in range(L)]
    _dnums = lax.GatherDimensionNumbers(
        offset_dims=(), collapsed_slice_dims=(0,), start_index_map=(0,))

    def _bcast_lane(vec, i):
        return lax.gather(vec, idx_splats[i], _dnums, (1,),
                          mode=lax.GatherScatterMode.PROMISE_IN_BOUNDS)

    def compute_chunk(xrow_v, out_v, chunk):
        @pl.loop(0, CHUNK_G // GQ)
        def _g_loop(gq):
            g_base = gq * GQ
            for q in range(GQ):
                g = g_base + q
                src = (chunk * CHUNK_G + g) * 32
                a_vec = xrow_v[pl.ds(src, L)]
                c_vec = xrow_v[pl.ds(src + L, L)]
                for i in range(L):
                    a_bcast = _bcast_lane(a_vec, i)
                    out_v[pl.ds(g * 256 + i * L, L)] = c_vec + a_bcast

    def do_row(r, xrow_v, xrow_next, isem, isem_next, skip_waits):
        row = row0 + r
        pltpu.make_async_copy(x_hbm.at[row], xrow_v, isem).wait()
        @pl.when(row + 1 < row0 + ROWS_PER_W)
        def _prefetch():
            pltpu.async_copy(x_hbm.at[row + 1], xrow_next, isem_next)

        for chunk in range(CHUNKS):
            ob = obs[chunk]
            dst = out_hbm.at[row, pl.ds(chunk * CHUNK_WORDS, CHUNK_WORDS)]
            if chunk not in skip_waits:
                pltpu.make_async_copy(ob, dst, osem).wait()
            compute_chunk(xrow_v, ob, chunk)
            pltpu.async_copy(ob, dst, osem)

    pltpu.async_copy(x_hbm.at[row0], xrow0, isem0)
    do_row(0, xrow0, xrow1, isem0, isem1, skip_waits=(0, 1, 2))
    do_row(1, xrow1, xrow0, isem1, isem0, skip_waits=())

    @pl.loop(1, ROWS_PER_W // 2)
    def _row_pair(rp):
        do_row(2 * rp, xrow0, xrow1, isem0, isem1, skip_waits=())
        do_row(2 * rp + 1, xrow1, xrow0, isem1, isem0, skip_waits=())

    for _ in range(NBUF - 1):
        pltpu.make_async_copy(
            ob0, out_hbm.at[row0, pl.ds(0, CHUNK_WORDS)], osem).wait()


@jax.jit
def kernel(input):
    mesh = plsc.VectorSubcoreMesh(core_axis_name="c", subcore_axis_name="s")
    run = pl.kernel(
        _sc_body,
        out_type=jax.ShapeDtypeStruct((ROWS, NUM_OUT), jnp.float32),
        mesh=mesh,
        scratch_types=[
            pltpu.VMEM((COLS,), jnp.float32),
            pltpu.VMEM((COLS,), jnp.float32),
            pltpu.VMEM((CHUNK_WORDS,), jnp.float32),
            pltpu.VMEM((CHUNK_WORDS,), jnp.float32),
            pltpu.VMEM((CHUNK_WORDS,), jnp.float32),
            pltpu.VMEM((CHUNK_WORDS,), jnp.float32),
            pltpu.SemaphoreType.DMA,
            pltpu.SemaphoreType.DMA,
            pltpu.SemaphoreType.DMA,
        ],
        compiler_params=_CPARAMS,
    )
    return run(input)
